# baseline (speedup 1.0000x reference)
import functools

import jax
import jax.numpy as jnp
from jax import lax
from jax.experimental import pallas as pl
from jax.experimental.pallas import tpu as pltpu
from jax.experimental.pallas import tpu_sc as plsc

_INFO = plsc.get_sparse_core_info()
_NC, _NS, _L = _INFO.num_cores, _INFO.num_subcores, _INFO.num_lanes
_NW = _NC * _NS


def _make_sc_kernel(n_rows, chunk, seq, dim):
    n_chunks_total = n_rows // chunk
    chunks_per_w = n_chunks_total // _NW
    pe_steps = seq // chunk
    mesh = plsc.VectorSubcoreMesh(core_axis_name="c", subcore_axis_name="s")

    @functools.partial(
        pl.kernel,
        mesh=mesh,
        compiler_params=pltpu.CompilerParams(use_tc_tiling_on_sc=False),
        out_type=jax.ShapeDtypeStruct((n_chunks_total, chunk, dim), jnp.float32),
        scratch_types=[
            pltpu.VMEM((chunks_per_w, chunk), jnp.int32),
            pltpu.VMEM((seq, dim), jnp.float32),
            pltpu.VMEM((chunk, dim), jnp.float32),
            pltpu.SemaphoreType.DMA,
        ],
    )
    def k(idx_hbm, pe_hbm, table_hbm, out_hbm, idx_v, pe_v, rows_v, sem):
        wid = lax.axis_index("s") * _NC + lax.axis_index("c")
        cbase = wid * chunks_per_w
        pltpu.sync_copy(idx_hbm.at[pl.ds(cbase, chunks_per_w)], idx_v)
        pltpu.sync_copy(pe_hbm, pe_v)

        def chunk_body(c, _):
            pltpu.async_copy(table_hbm.at[idx_v.at[c]], rows_v, sem).wait()
            po = lax.rem(c, pe_steps) * chunk

            def row_body(r, _):
                for j in range(dim // _L):
                    s = pl.ds(j * _L, _L)
                    rows_v[r, s] = rows_v[r, s] + pe_v[po + r, s]
                return 0

            lax.fori_loop(0, chunk, row_body, 0, unroll=2)
            pltpu.sync_copy(rows_v, out_hbm.at[cbase + c])
            return 0

        lax.fori_loop(0, chunks_per_w, chunk_body, 0)

    return k


def kernel(x, table):
    b, s = x.shape
    v, d = table.shape
    n_rows = b * s
    chunk = 100

    pos = jnp.arange(s, dtype=jnp.float32)
    pe = jnp.tile(jnp.stack([jnp.sin(pos), jnp.cos(pos)], axis=1), (1, d // 2))

    idx = x.reshape(n_rows // chunk, chunk).astype(jnp.int32)
    out = _make_sc_kernel(n_rows, chunk, s, d)(idx, pe, table)
    return out.reshape(b, s, d)


# speedup vs baseline: 1.0743x; 1.0743x over previous
import functools

import jax
import jax.numpy as jnp
from jax import lax
from jax.experimental import pallas as pl
from jax.experimental.pallas import tpu as pltpu
from jax.experimental.pallas import tpu_sc as plsc

_INFO = plsc.get_sparse_core_info()
_NC, _NS, _L = _INFO.num_cores, _INFO.num_subcores, _INFO.num_lanes
_NW = _NC * _NS


def _make_sc_kernel(n_rows, chunk, seq, dim):
    n_chunks_total = n_rows // chunk
    chunks_per_w = n_chunks_total // _NW
    pe_steps = seq // chunk
    mesh = plsc.VectorSubcoreMesh(core_axis_name="c", subcore_axis_name="s")

    half = chunks_per_w // 2

    @functools.partial(
        pl.kernel,
        mesh=mesh,
        compiler_params=pltpu.CompilerParams(use_tc_tiling_on_sc=False),
        out_type=jax.ShapeDtypeStruct((n_chunks_total, chunk, dim), jnp.float32),
        scratch_types=[
            pltpu.VMEM((chunks_per_w, chunk), jnp.int32),
            pltpu.VMEM((seq, dim), jnp.float32),
            pltpu.VMEM((chunk, dim), jnp.float32),
            pltpu.VMEM((chunk, dim), jnp.float32),
            pltpu.VMEM((chunk, dim), jnp.float32),
            pltpu.VMEM((chunk, dim), jnp.float32),
            pltpu.SemaphoreType.DMA,
            pltpu.SemaphoreType.DMA,
            pltpu.SemaphoreType.DMA,
            pltpu.SemaphoreType.DMA,
        ],
    )
    def k(idx_hbm, pe_hbm, table_hbm, out_hbm,
          idx_v, pe_v, g0, g1, s0, s1, gs0, gs1, ss0, ss1):
        wid = lax.axis_index("s") * _NC + lax.axis_index("c")
        cbase = wid * chunks_per_w
        pltpu.sync_copy(idx_hbm.at[pl.ds(cbase, chunks_per_w)], idx_v)
        pltpu.sync_copy(pe_hbm, pe_v)

        def gather_start(c, buf, sem):
            pltpu.async_copy(table_hbm.at[idx_v.at[c]], buf, sem)

        def gather_wait(buf, sem):
            pltpu.make_async_copy(table_hbm.at[idx_v.at[0]], buf, sem).wait()

        def store_wait(buf, sem):
            pltpu.make_async_copy(buf, out_hbm.at[cbase], sem).wait()

        def add(src, dst, po):
            def row_body(r, _):
                for j in range(dim // _L):
                    s = pl.ds(j * _L, _L)
                    dst[r, s] = src[r, s] + pe_v[po + r, s]
                return 0

            lax.fori_loop(0, chunk, row_body, 0, unroll=2)

        gather_start(0, g0, gs0)
        gather_start(1, g1, gs1)

        def pair_body(i, _):
            for b, (g, s, gs, ss) in enumerate(
                ((g0, s0, gs0, ss0), (g1, s1, gs1, ss1))):
                c = 2 * i + b
                gather_wait(g, gs)

                @pl.when(i >= 1)
                def _():
                    store_wait(s, ss)

                add(g, s, lax.rem(c, pe_steps) * chunk)

                @pl.when(i < half - 1)
                def _():
                    gather_start(c + 2, g, gs)

                pltpu.async_copy(s, out_hbm.at[cbase + c], ss)
            return 0

        lax.fori_loop(0, half, pair_body, 0)
        store_wait(s0, ss0)
        store_wait(s1, ss1)

    return k


def kernel(x, table):
    b, s = x.shape
    v, d = table.shape
    n_rows = b * s
    chunk = 100

    pos = jnp.arange(s, dtype=jnp.float32)
    pe = jnp.tile(jnp.stack([jnp.sin(pos), jnp.cos(pos)], axis=1), (1, d // 2))

    idx = x.reshape(n_rows // chunk, chunk).astype(jnp.int32)
    out = _make_sc_kernel(n_rows, chunk, s, d)(idx, pe, table)
    return out.reshape(b, s, d)


# speedup vs baseline: 1.1840x; 1.1022x over previous
import functools

import jax
import jax.numpy as jnp
from jax import lax
from jax.experimental import pallas as pl
from jax.experimental.pallas import tpu as pltpu
from jax.experimental.pallas import tpu_sc as plsc

_INFO = plsc.get_sparse_core_info()
_NC, _NS, _L = _INFO.num_cores, _INFO.num_subcores, _INFO.num_lanes
_NW = _NC * _NS


def _make_sc_kernel(n_rows, chunk, seq, dim):
    n_chunks_total = n_rows // chunk
    chunks_per_w = n_chunks_total // _NW
    pe_steps = seq // chunk
    mesh = plsc.VectorSubcoreMesh(core_axis_name="c", subcore_axis_name="s")

    half = chunks_per_w // 2

    @functools.partial(
        pl.kernel,
        mesh=mesh,
        compiler_params=pltpu.CompilerParams(use_tc_tiling_on_sc=False),
        out_type=jax.ShapeDtypeStruct((n_chunks_total, chunk, dim), jnp.float32),
        scratch_types=[
            pltpu.VMEM((chunks_per_w, chunk), jnp.int32),
            pltpu.VMEM((seq, dim), jnp.float32),
            pltpu.VMEM((chunk, dim), jnp.float32),
            pltpu.VMEM((chunk, dim), jnp.float32),
            pltpu.VMEM((chunk, dim), jnp.float32),
            pltpu.VMEM((chunk, dim), jnp.float32),
            pltpu.SemaphoreType.DMA,
            pltpu.SemaphoreType.DMA,
            pltpu.SemaphoreType.DMA,
            pltpu.SemaphoreType.DMA,
        ],
    )
    def k(idx_hbm, pe_hbm, table_hbm, out_hbm,
          idx_v, pe_v, g0, g1, s0, s1, gs0, gs1, ss0, ss1):
        wid = lax.axis_index("s") * _NC + lax.axis_index("c")
        cbase = wid * chunks_per_w
        pltpu.sync_copy(idx_hbm.at[pl.ds(cbase, chunks_per_w)], idx_v)
        pltpu.sync_copy(pe_hbm, pe_v)

        def gather_start(c, buf, sem):
            pltpu.async_copy(table_hbm.at[idx_v.at[c]], buf, sem)

        def gather_wait(buf, sem):
            pltpu.make_async_copy(table_hbm.at[idx_v.at[0]], buf, sem).wait()

        def store_wait(buf, sem):
            pltpu.make_async_copy(buf, out_hbm.at[cbase], sem).wait()

        def add(src, dst, po):
            @plsc.parallel_loop(0, chunk, unroll=4)
            def _(r):
                for j in range(dim // _L):
                    s = pl.ds(j * _L, _L)
                    dst[r, s] = src[r, s] + pe_v[po + r, s]

        gather_start(0, g0, gs0)
        gather_start(1, g1, gs1)

        def pair_body(i, _):
            for b, (g, s, gs, ss) in enumerate(
                ((g0, s0, gs0, ss0), (g1, s1, gs1, ss1))):
                c = 2 * i + b
                gather_wait(g, gs)

                @pl.when(i >= 1)
                def _():
                    store_wait(s, ss)

                add(g, s, lax.rem(c, pe_steps) * chunk)

                @pl.when(i < half - 1)
                def _():
                    gather_start(c + 2, g, gs)

                pltpu.async_copy(s, out_hbm.at[cbase + c], ss)
            return 0

        lax.fori_loop(0, half, pair_body, 0)
        store_wait(s0, ss0)
        store_wait(s1, ss1)

    return k


def kernel(x, table):
    b, s = x.shape
    v, d = table.shape
    n_rows = b * s
    chunk = 100

    pos = jnp.arange(s, dtype=jnp.float32)
    pe = jnp.tile(jnp.stack([jnp.sin(pos), jnp.cos(pos)], axis=1), (1, d // 2))

    idx = x.reshape(n_rows // chunk, chunk).astype(jnp.int32)
    out = _make_sc_kernel(n_rows, chunk, s, d)(idx, pe, table)
    return out.reshape(b, s, d)
